# baseline (speedup 1.0000x reference)
import functools

import jax
import jax.numpy as jnp
from jax.experimental import pallas as pl
from jax.experimental.pallas import tpu as pltpu

B, S, E, N1, N2 = 16, 256, 256, 8, 8
KV, QD = 100, 768
E_BLK = 64
EB = E // E_BLK


def _prep_body(q0_ref, wq2t_ref, bq2_ref, wk2_ref, wq1t_ref, bq1_ref, wk1_ref,
               qk2_ref, qk1_ref):
    q0 = q0_ref[...]
    qi2 = jnp.tanh(jnp.dot(q0, wq2t_ref[...]) + bq2_ref[...])
    qk2 = jnp.dot(qi2, wk2_ref[...]) * 0.1
    qi1 = jnp.tanh(jnp.dot(q0, wq1t_ref[...]) + bq1_ref[...])
    qk1 = jnp.dot(qi1, wk1_ref[...]) * 0.1
    qk2_ref[...] = qk2[:, None, :]
    qk1_ref[...] = qk1[:, None, :]


def _main_body(ient_ref, qk2_ref, qk1_ref, k2_ref, v2_ref, k1_ref, v1_ref,
               out_ref, c1_ref):
    eb = pl.program_id(1)

    qv2 = qk2_ref[0]
    qv1 = qk1_ref[0]

    k2 = k2_ref[0]
    v2 = v2_ref[0]
    logits2 = jnp.sum(k2 * qv2[None], axis=-1, keepdims=True)
    e2 = jnp.exp(logits2)
    attn2 = e2 / jnp.sum(e2, axis=1, keepdims=True)
    comb = jnp.sum(attn2 * v2, axis=1)
    comb3 = comb.reshape(E_BLK, N1, KV)

    k1 = k1_ref[0]
    v1 = v1_ref[0]
    logits1 = jnp.sum(k1 * qv1[None], axis=-1, keepdims=True)
    e1 = jnp.exp(logits1)
    attn1 = e1 / jnp.sum(e1, axis=1, keepdims=True)
    o_a = jnp.sum(attn1 * v1, axis=1)
    o_b = jnp.sum(attn1 * comb3, axis=1)
    c1_ref[pl.ds(eb * E_BLK, E_BLK), :] = jnp.concatenate([o_a, o_b], axis=-1)

    @pl.when(eb == EB - 1)
    def _assemble():
        mask_col = (ient_ref[0] != 0).astype(jnp.float32)
        s_iota = jax.lax.broadcasted_iota(jnp.int32, (S, S), 0)
        t_iota = jax.lax.broadcasted_iota(jnp.int32, (S, S), 1)
        tril = (t_iota <= s_iota).astype(jnp.float32)
        csum = jnp.dot(tril, mask_col)
        order = jnp.clip(csum - 1.0, 0.0, float(E - 1))
        sel = (order == t_iota.astype(jnp.float32)).astype(jnp.float32) \
            * mask_col
        out_ref[0] = jnp.dot(sel, c1_ref[...])


@functools.partial(jax.jit, static_argnames=("interpret",))
def _run(input_ent, q, k_hop1, v_hop1, k_hop2, v_hop2, w_q2, b_q2, w_k2,
         w_q1, b_q1, w_k1, interpret=False):
    f32 = jnp.float32
    q0 = q[:, 0, :]
    qk2, qk1 = pl.pallas_call(
        _prep_body,
        out_shape=(jax.ShapeDtypeStruct((B, 1, KV), f32),
                   jax.ShapeDtypeStruct((B, 1, KV), f32)),
        interpret=interpret,
    )(q0, w_q2.T, b_q2.reshape(1, KV), w_k2, w_q1.T, b_q1.reshape(1, KV), w_k1)

    ient = input_ent.astype(jnp.int32).reshape(B, S, 1)
    k2r = k_hop2.reshape(B, E * N1, N2, KV)
    v2r = v_hop2.reshape(B, E * N1, N2, KV)

    grid = (B, EB)
    out = pl.pallas_call(
        _main_body,
        grid=grid,
        in_specs=[
            pl.BlockSpec((1, S, 1), lambda b, e: (b, 0, 0)),
            pl.BlockSpec((1, 1, KV), lambda b, e: (b, 0, 0)),
            pl.BlockSpec((1, 1, KV), lambda b, e: (b, 0, 0)),
            pl.BlockSpec((1, E_BLK * N1, N2, KV), lambda b, e: (b, e, 0, 0)),
            pl.BlockSpec((1, E_BLK * N1, N2, KV), lambda b, e: (b, e, 0, 0)),
            pl.BlockSpec((1, E_BLK, N1, KV), lambda b, e: (b, e, 0, 0)),
            pl.BlockSpec((1, E_BLK, N1, KV), lambda b, e: (b, e, 0, 0)),
        ],
        out_specs=pl.BlockSpec((1, S, 2 * KV), lambda b, e: (b, 0, 0)),
        out_shape=jax.ShapeDtypeStruct((B, S, 2 * KV), f32),
        scratch_shapes=[pltpu.VMEM((E, 2 * KV), f32)],
        compiler_params=pltpu.CompilerParams(
            dimension_semantics=("parallel", "arbitrary"),
        ),
        interpret=interpret,
    )(ient, qk2, qk1, k2r, v2r, k_hop1, v_hop1)
    return out


def kernel(input_ent, q, k_hop1, v_hop1, k_hop2, v_hop2, w_q2, b_q2, w_k2,
           w_q1, b_q1, w_k1):
    return _run(input_ent, q, k_hop1, v_hop1, k_hop2, v_hop2, w_q2, b_q2,
                w_k2, w_q1, b_q1, w_k1)
